# Initial kernel scaffold; baseline (speedup 1.0000x reference)
import jax
import jax.numpy as jnp
from jax.experimental import pallas as pl


def kernel(permute, src):
    raise NotImplementedError("write your pallas kernel here")


# speedup vs baseline: 1.3295x; 1.3295x over previous
import functools

import jax
import jax.numpy as jnp
from jax import lax
from jax.experimental import pallas as pl
from jax.experimental.pallas import tpu as pltpu
from jax.experimental.pallas import tpu_sc as plsc

S, B, D = 2048, 4, 2048
NW = 32
S_PER_W = S // (NW // B)
K = 16
NCH = S_PER_W // K

_mesh = plsc.VectorSubcoreMesh(core_axis_name="c", subcore_axis_name="s")


@functools.partial(
    pl.kernel,
    mesh=_mesh,
    out_type=jax.ShapeDtypeStruct((S, B, D), jnp.float32),
    scratch_types=[
        pltpu.VMEM((S_PER_W,), jnp.int32),
        pltpu.VMEM((NCH, K), jnp.int32),
        pltpu.VMEM((K, D), jnp.float32),
        pltpu.VMEM((K, D), jnp.float32),
        pltpu.SemaphoreType.DMA,
        pltpu.SemaphoreType.DMA,
        pltpu.SemaphoreType.DMA,
        pltpu.SemaphoreType.DMA,
    ],
)
def _sc_gather(perm_hbm, srcf_hbm, out_hbm, perm_v, idx2d, buf0, buf1,
               sg0, sg1, ss0, ss1):
    cid = lax.axis_index("c")
    sid = lax.axis_index("s")
    wid = sid * 2 + cid
    b = wid % B
    s0 = (wid // B) * S_PER_W

    pltpu.sync_copy(perm_hbm.at[b, pl.ds(s0, S_PER_W)], perm_v)

    for i in range(NCH):
        idx2d[i, :] = perm_v[pl.ds(i * K, K)] * B + b

    bufs = (buf0, buf1)
    gsems = (sg0, sg1)
    ssems = (ss0, ss1)

    def gather(c):
        p = c % 2
        return pltpu.make_async_copy(
            srcf_hbm.at[idx2d.at[c]], bufs[p], gsems[p])

    def store(c):
        p = c % 2
        return pltpu.make_async_copy(
            bufs[p], out_hbm.at[pl.ds(s0 + c * K, K), b], ssems[p])

    gather(0).start()
    for c in range(NCH):
        nxt = c + 1
        if nxt < NCH:
            if nxt >= 2:
                store(nxt - 2).wait()
            gather(nxt).start()
        gather(c).wait()
        store(c).start()
    store(NCH - 2).wait()
    store(NCH - 1).wait()


def kernel(permute, src):
    src_flat = src.reshape(S * B, D)
    return _sc_gather(permute, src_flat)


# speedup vs baseline: 2.6172x; 1.9685x over previous
import functools

import jax
import jax.numpy as jnp
from jax import lax
from jax.experimental import pallas as pl
from jax.experimental.pallas import tpu as pltpu
from jax.experimental.pallas import tpu_sc as plsc

S, B, D = 2048, 4, 2048
NW = 32
S_PER_W = S // (NW // B)
K = 16
NCH = S_PER_W // K

_mesh = plsc.VectorSubcoreMesh(core_axis_name="c", subcore_axis_name="s")


@functools.partial(
    pl.kernel,
    mesh=_mesh,
    out_type=jax.ShapeDtypeStruct((S, B, D), jnp.float32),
    scratch_types=[
        pltpu.VMEM((S_PER_W,), jnp.int32),
        pltpu.VMEM((NCH, K), jnp.int32),
        pltpu.VMEM((K, 1, D), jnp.float32),
        pltpu.VMEM((K, 1, D), jnp.float32),
        pltpu.SemaphoreType.DMA,
        pltpu.SemaphoreType.DMA,
        pltpu.SemaphoreType.DMA,
        pltpu.SemaphoreType.DMA,
    ],
)
def _sc_gather(perm_hbm, srcf_hbm, out_hbm, perm_v, idx2d, buf0, buf1,
               sg0, sg1, ss0, ss1):
    cid = lax.axis_index("c")
    sid = lax.axis_index("s")
    wid = sid * 2 + cid
    b = wid % B
    s0 = (wid // B) * S_PER_W

    pltpu.sync_copy(perm_hbm.at[b, pl.ds(s0, S_PER_W)], perm_v)

    for i in range(NCH):
        idx2d[i, :] = perm_v[pl.ds(i * K, K)]

    bufs = (buf0, buf1)
    gsems = (sg0, sg1)
    ssems = (ss0, ss1)

    def gather(c):
        p = c % 2
        return pltpu.make_async_copy(
            srcf_hbm.at[idx2d.at[c], pl.ds(b, 1)], bufs[p], gsems[p])

    def store(c):
        p = c % 2
        return pltpu.make_async_copy(
            bufs[p], out_hbm.at[pl.ds(s0 + c * K, K), pl.ds(b, 1)], ssems[p])

    gather(0).start()
    for c in range(NCH):
        nxt = c + 1
        if nxt < NCH:
            if nxt >= 2:
                store(nxt - 2).wait()
            gather(nxt).start()
        gather(c).wait()
        store(c).start()
    store(NCH - 2).wait()
    store(NCH - 1).wait()


def kernel(permute, src):
    return _sc_gather(permute, src)


# speedup vs baseline: 2.7283x; 1.0424x over previous
import functools

import jax
import jax.numpy as jnp
from jax import lax
from jax.experimental import pallas as pl
from jax.experimental.pallas import tpu as pltpu
from jax.experimental.pallas import tpu_sc as plsc

S, B, D = 2048, 4, 2048
NW = 32
S_PER_W = S // (NW // B)
K = 24
CK = [K] * 10 + [16]
NCH = len(CK)
CS = [sum(CK[:i]) for i in range(NCH)]

_mesh = plsc.VectorSubcoreMesh(core_axis_name="c", subcore_axis_name="s")


@functools.partial(
    pl.kernel,
    mesh=_mesh,
    out_type=jax.ShapeDtypeStruct((S, B, D), jnp.float32),
    scratch_types=[
        pltpu.VMEM((S_PER_W,), jnp.int32),
        pltpu.VMEM((NCH, 32), jnp.int32),
        pltpu.VMEM((K, 1, D), jnp.float32),
        pltpu.VMEM((K, 1, D), jnp.float32),
        pltpu.SemaphoreType.DMA,
        pltpu.SemaphoreType.DMA,
        pltpu.SemaphoreType.DMA,
        pltpu.SemaphoreType.DMA,
    ],
)
def _sc_gather(perm_hbm, srcf_hbm, out_hbm, perm_v, idx2d, buf0, buf1,
               sg0, sg1, ss0, ss1):
    cid = lax.axis_index("c")
    sid = lax.axis_index("s")
    wid = sid * 2 + cid
    b = wid % B
    s0 = (wid // B) * S_PER_W

    pltpu.sync_copy(perm_hbm.at[b, pl.ds(s0, S_PER_W)], perm_v)

    for i in range(NCH):
        for j in (0, 16):
            if j < CK[i] and CS[i] + j + 16 <= S_PER_W:
                idx2d[i, pl.ds(j, 16)] = perm_v[pl.ds(CS[i] + j, 16)]

    bufs = (buf0, buf1)
    gsems = (sg0, sg1)
    ssems = (ss0, ss1)

    def gather(c):
        p = c % 2
        return pltpu.make_async_copy(
            srcf_hbm.at[idx2d.at[c, pl.ds(0, CK[c])], pl.ds(b, 1)],
            bufs[p].at[pl.ds(0, CK[c])], gsems[p])

    def store(c):
        p = c % 2
        return pltpu.make_async_copy(
            bufs[p].at[pl.ds(0, CK[c])],
            out_hbm.at[pl.ds(s0 + CS[c], CK[c]), pl.ds(b, 1)], ssems[p])

    gather(0).start()
    for c in range(NCH):
        nxt = c + 1
        if nxt < NCH:
            if nxt >= 2:
                store(nxt - 2).wait()
            gather(nxt).start()
        gather(c).wait()
        store(c).start()
    store(NCH - 2).wait()
    store(NCH - 1).wait()


def kernel(permute, src):
    return _sc_gather(permute, src)


# speedup vs baseline: 2.7321x; 1.0014x over previous
import functools

import jax
import jax.numpy as jnp
from jax import lax
from jax.experimental import pallas as pl
from jax.experimental.pallas import tpu as pltpu
from jax.experimental.pallas import tpu_sc as plsc

S, B, D = 2048, 4, 2048
NW = 32
S_PER_W = S // (NW // B)
K = 16
CK = [8, 8] + [16] * 15
NCH = len(CK)
CS = [sum(CK[:i]) for i in range(NCH)]

_mesh = plsc.VectorSubcoreMesh(core_axis_name="c", subcore_axis_name="s")


@functools.partial(
    pl.kernel,
    mesh=_mesh,
    out_type=jax.ShapeDtypeStruct((S, B, D), jnp.float32),
    scratch_types=[
        pltpu.VMEM((S_PER_W,), jnp.int32),
        pltpu.VMEM((NCH, 32), jnp.int32),
        pltpu.VMEM((K, 1, D), jnp.float32),
        pltpu.VMEM((K, 1, D), jnp.float32),
        pltpu.VMEM((K, 1, D), jnp.float32),
        pltpu.SemaphoreType.DMA,
        pltpu.SemaphoreType.DMA,
        pltpu.SemaphoreType.DMA,
        pltpu.SemaphoreType.DMA,
        pltpu.SemaphoreType.DMA,
        pltpu.SemaphoreType.DMA,
    ],
)
def _sc_gather(perm_hbm, srcf_hbm, out_hbm, perm_v, idx2d, buf0, buf1, buf2,
               sg0, sg1, sg2, ss0, ss1, ss2):
    cid = lax.axis_index("c")
    sid = lax.axis_index("s")
    wid = sid * 2 + cid
    b = wid % B
    s0 = (wid // B) * S_PER_W

    pltpu.sync_copy(perm_hbm.at[b, pl.ds(s0, S_PER_W)], perm_v)

    for i in range(NCH):
        for j in (0, 16):
            if j < CK[i] and CS[i] + j + 16 <= S_PER_W:
                idx2d[i, pl.ds(j, 16)] = perm_v[pl.ds(CS[i] + j, 16)]

    bufs = (buf0, buf1, buf2)
    gsems = (sg0, sg1, sg2)
    ssems = (ss0, ss1, ss2)

    def gather(c):
        p = c % 3
        return pltpu.make_async_copy(
            srcf_hbm.at[idx2d.at[c, pl.ds(0, CK[c])], pl.ds(b, 1)],
            bufs[p].at[pl.ds(0, CK[c])], gsems[p])

    def store(c):
        p = c % 3
        return pltpu.make_async_copy(
            bufs[p].at[pl.ds(0, CK[c])],
            out_hbm.at[pl.ds(s0 + CS[c], CK[c]), pl.ds(b, 1)], ssems[p])

    gather(0).start()
    gather(1).start()
    for c in range(NCH):
        g = c + 2
        if g < NCH:
            if g >= 3:
                store(g - 3).wait()
            gather(g).start()
        gather(c).wait()
        store(c).start()
    store(NCH - 3).wait()
    store(NCH - 2).wait()
    store(NCH - 1).wait()


def kernel(permute, src):
    return _sc_gather(permute, src)
